# baseline (speedup 1.0000x reference)
import functools
import math

import jax
import jax.numpy as jnp
import numpy as np
from jax import lax
from jax.experimental import pallas as pl
from jax.experimental.pallas import tpu as pltpu
from jax.experimental.pallas import tpu_sc as plsc

POS_DIM = 3
CAPACITY = 2 ** 18
NR_LEVELS = 24
NR_FEAT = 2
GEOM_FEAT = 32
NR_ITERS_C2F = 10000
SCALES = np.geomspace(1.0, 0.0001, NR_LEVELS).astype(np.float32)
P0 = np.int32(np.uint32(2654435761).astype(np.int64) - (1 << 32))
P1 = np.int32(805459861)
P2 = np.int32(np.uint32(3674653429).astype(np.int64) - (1 << 32))
SF = [(POS_DIM + 1) / math.sqrt((i + 1) * (i + 2)) for i in range(POS_DIM)]

N_POINTS = 200000
LANES = 128
N_PAD = 200704
R_ROWS = N_PAD // LANES
RB = 8
NV = 96
SC_TILES = 32
PT_PER_TILE = N_PAD // SC_TILES
SC_CHUNK = 128
SC_NCHUNKS = PT_PER_TILE // SC_CHUNK
B3 = 2048


def _lattice_body(shifts_ref, window_ref, x_ref, y_ref, z_ref, idx_ref, w_ref):
    x = x_ref[...]
    y = y_ref[...]
    z = z_ref[...]
    for l in range(NR_LEVELS):
        scale = float(SCALES[l])
        xs0 = (x + shifts_ref[l, 0]) / scale * SF[0]
        xs1 = (y + shifts_ref[l, 1]) / scale * SF[1]
        xs2 = (z + shifts_ref[l, 2]) / scale * SF[2]
        e3 = -3.0 * xs2
        sm = xs2
        e2 = sm - 2.0 * xs1
        sm = sm + xs1
        e1 = sm - xs0
        sm = sm + xs0
        e0 = sm
        elev = [e0, e1, e2, e3]
        rem0 = [jnp.floor(e / 4.0 + 0.5) * 4.0 for e in elev]
        diff = [elev[j] - rem0[j] for j in range(4)]
        rank = [jnp.zeros_like(x, jnp.int32) for _ in range(4)]
        for i in range(4):
            for j in range(i + 1, 4):
                c = (diff[i] < diff[j]).astype(jnp.int32)
                rank[i] = rank[i] + c
                rank[j] = rank[j] + (1 - c)
        ssum = ((rem0[0] + rem0[1]) + (rem0[2] + rem0[3])) / 4.0
        s_i = jnp.floor(ssum + 0.5).astype(jnp.int32)
        for j in range(4):
            rank[j] = rank[j] + s_i
            low = (rank[j] < 0).astype(jnp.int32)
            high = (rank[j] > 3).astype(jnp.int32)
            rank[j] = rank[j] + 4 * low - 4 * high
            rem0[j] = rem0[j] + 4.0 * (low - high).astype(jnp.float32)
        v = [(elev[j] - rem0[j]) / 4.0 for j in range(4)]
        bc = []
        for k in range(5):
            acc = jnp.zeros_like(x)
            for j in range(4):
                acc = acc + v[j] * ((rank[j] == 3 - k).astype(jnp.float32)
                                    - (rank[j] == 4 - k).astype(jnp.float32))
            bc.append(acc)
        bc[0] = bc[0] + (1.0 + bc[4])
        rem0i = [rem0[j].astype(jnp.int32) for j in range(3)]
        win = window_ref[l]
        for k in range(4):
            key0 = rem0i[0] + k - 4 * (rank[0] > 3 - k).astype(jnp.int32)
            key1 = rem0i[1] + k - 4 * (rank[1] > 3 - k).astype(jnp.int32)
            key2 = rem0i[2] + k - 4 * (rank[2] > 3 - k).astype(jnp.int32)
            h = (key0 * P0) ^ (key1 * P1) ^ (key2 * P2)
            idx = ((h & jnp.int32(CAPACITY - 1)) + jnp.int32(l * CAPACITY)) * 2
            idx_ref[0, l * 4 + k] = idx
            w_ref[0, l * 4 + k] = bc[k] * win


def _phase1(shifts, window, xr, yr, zr):
    smem = pl.BlockSpec(memory_space=pltpu.SMEM)
    blk = pl.BlockSpec((RB, LANES), lambda i: (i, 0))
    return pl.pallas_call(
        _lattice_body,
        grid=(R_ROWS // RB,),
        in_specs=[smem, smem, blk, blk, blk],
        out_specs=[pl.BlockSpec((1, NV, RB, LANES), lambda i: (i, 0, 0, 0)),
                   pl.BlockSpec((1, NV, RB, LANES), lambda i: (i, 0, 0, 0))],
        out_shape=[
            jax.ShapeDtypeStruct((R_ROWS // RB, NV, RB, LANES), jnp.int32),
            jax.ShapeDtypeStruct((R_ROWS // RB, NV, RB, LANES), jnp.float32)],
    )(shifts, window, xr, yr, zr)


def _sc_gather_body(table_hbm1d, idx_hbm, w_hbm, out_hbm,
                    idx_v, odd_v, w_v, rows_a_v, rows_b_v, feats_v, sem):
    wid = lax.axis_index("s") * 2 + lax.axis_index("c")
    lane = lax.iota(jnp.int32, 16)

    def chunk_body(g, carry):
        cg = wid * SC_NCHUNKS + g
        base = cg * SC_CHUNK
        i_blk = cg // RB
        s_sub = cg % RB
        row0 = i_blk * (NV * RB * LANES) + s_sub * LANES

        def slab_fire(r, c2):
            off = row0 + r * (RB * LANES)
            pltpu.async_copy(idx_hbm.at[pl.ds(off, SC_CHUNK)],
                             idx_v.at[r], sem)
            pltpu.async_copy(w_hbm.at[pl.ds(off, SC_CHUNK)],
                             w_v.at[r], sem)
            return c2

        lax.fori_loop(0, NV, slab_fire, 0)

        def slab_drain(r, c2):
            pltpu.make_async_copy(idx_hbm.at[pl.ds(0, SC_CHUNK)],
                                  idx_v.at[0], sem).wait()
            pltpu.make_async_copy(w_hbm.at[pl.ds(0, SC_CHUNK)],
                                  w_v.at[0], sem).wait()
            return c2

        lax.fori_loop(0, NV, slab_drain, 0)

        def mk_odd(r, c2):
            for v in range(SC_CHUNK // 16):
                odd_v[r, pl.ds(v * 16, 16)] = idx_v[r, pl.ds(v * 16, 16)] + 1
            return c2

        lax.fori_loop(0, NV, mk_odd, 0)

        def fire(r, c2):
            pltpu.async_copy(table_hbm1d.at[idx_v.at[r]], rows_a_v.at[r], sem)
            pltpu.async_copy(table_hbm1d.at[odd_v.at[r]], rows_b_v.at[r], sem)
            return c2

        lax.fori_loop(0, NV, fire, 0)

        def drain(r, c2):
            pltpu.make_async_copy(table_hbm1d.at[idx_v.at[0]],
                                  rows_a_v.at[0], sem).wait()
            pltpu.make_async_copy(table_hbm1d.at[idx_v.at[0]],
                                  rows_b_v.at[0], sem).wait()
            return c2

        lax.fori_loop(0, NV, drain, 0)

        def level_body(l, carry2):
            for pv in range(SC_CHUNK // 16):
                p0 = pv * 16
                for c in range(2):
                    rows_cv = rows_a_v if c == 0 else rows_b_v
                    acc = jnp.zeros((16,), jnp.float32)
                    for k in range(4):
                        r = 4 * l + k
                        wk = w_v[r, pl.ds(p0, 16)]
                        rvals = rows_cv[r, pl.ds(p0, 16)]
                        acc = acc + wk * rvals
                    feats_v[2 * l + c, pl.ds(p0, 16)] = acc
            return carry2

        lax.fori_loop(0, NR_LEVELS, level_body, 0)

        def out_fire(f, c2):
            pltpu.async_copy(feats_v.at[f],
                             out_hbm.at[pl.ds(f * N_PAD + base, SC_CHUNK)], sem)
            return c2

        lax.fori_loop(0, 2 * NR_LEVELS, out_fire, 0)

        def out_drain(f, c2):
            pltpu.make_async_copy(feats_v.at[0],
                                  out_hbm.at[pl.ds(0, SC_CHUNK)], sem).wait()
            return c2

        lax.fori_loop(0, 2 * NR_LEVELS, out_drain, 0)
        return carry

    lax.fori_loop(0, SC_NCHUNKS, chunk_body, 0)


def _phase2(table2, idx_all, w_all):
    mesh = plsc.VectorSubcoreMesh(core_axis_name="c", subcore_axis_name="s")
    kern = functools.partial(
        pl.kernel,
        out_type=jax.ShapeDtypeStruct((N_PAD * 2 * NR_LEVELS,), jnp.float32),
        mesh=mesh,
        scratch_types=[
            pltpu.VMEM((NV, SC_CHUNK), jnp.int32),
            pltpu.VMEM((NV, SC_CHUNK), jnp.int32),
            pltpu.VMEM((NV, SC_CHUNK), jnp.float32),
            pltpu.VMEM((NV, SC_CHUNK), jnp.float32),
            pltpu.VMEM((NV, SC_CHUNK), jnp.float32),
            pltpu.VMEM((2 * NR_LEVELS, SC_CHUNK), jnp.float32),
            pltpu.SemaphoreType.DMA,
        ],
        compiler_params=pltpu.CompilerParams(needs_layout_passes=False,
                                             use_tc_tiling_on_sc=False),
    )(_sc_gather_body)
    return kern(table2, idx_all, w_all)


def _gelu(x):
    return 0.5 * x * (1.0 + lax.erf(x * np.float32(1.0 / math.sqrt(2.0))))


RB3 = 16


def _mlp_body(feats_ref, pts_ref, w0f_ref, w0p_ref, b0_ref, w1_ref, b1_ref,
              w2_ref, b2_ref, w3_ref, b3_ref, out_ref):
    w0f = w0f_ref[...]
    w0p = w0p_ref[...]
    w1 = w1_ref[...]
    w2 = w2_ref[...]
    w3 = w3_ref[...]
    for j in range(RB3):
        f = feats_ref[:, j, :]
        p = pts_ref[:, j, :] * 0.001
        h = jnp.dot(w0f, f, preferred_element_type=jnp.float32)
        h = h + jnp.dot(w0p, p, preferred_element_type=jnp.float32)
        h = _gelu(h + b0_ref[...])
        h = _gelu(jnp.dot(w1, h, preferred_element_type=jnp.float32)
                  + b1_ref[...])
        h = _gelu(jnp.dot(w2, h, preferred_element_type=jnp.float32)
                  + b2_ref[...])
        out_ref[:, j, :] = (jnp.dot(w3, h, preferred_element_type=jnp.float32)
                            + b3_ref[...])


def _phase3(feats3, pts3, W0, b0, W1, b1, W2, b2, W3, b3):
    def whole(shape):
        return pl.BlockSpec(shape, lambda i: (0, 0))
    nf = 2 * NR_LEVELS
    no = 1 + GEOM_FEAT
    return pl.pallas_call(
        _mlp_body,
        grid=(R_ROWS // RB3,),
        in_specs=[pl.BlockSpec((nf, RB3, LANES), lambda i: (0, i, 0)),
                  pl.BlockSpec((POS_DIM, RB3, LANES), lambda i: (0, i, 0)),
                  whole((32, nf)), whole((32, POS_DIM)),
                  whole((32, 1)), whole((32, 32)), whole((32, 1)),
                  whole((32, 32)), whole((32, 1)),
                  whole((no, 32)), whole((no, 1))],
        out_specs=pl.BlockSpec((no, RB3, LANES), lambda i: (0, i, 0)),
        out_shape=jax.ShapeDtypeStruct((no, R_ROWS, LANES), jnp.float32),
    )(feats3, pts3,
      W0[:nf].T, W0[nf:].T, b0.reshape(-1, 1),
      W1.T, b1.reshape(-1, 1), W2.T, b2.reshape(-1, 1),
      W3.T, b3.reshape(-1, 1))


def _c2f_window(iter_nr):
    t = jnp.clip(jnp.asarray(iter_nr, jnp.float32), 0.0, float(NR_ITERS_C2F))
    tt = 0.3 + (1.0 - 0.3) * t / float(NR_ITERS_C2F)
    alpha = tt * NR_LEVELS
    k = jnp.arange(NR_LEVELS, dtype=jnp.float32)
    return 0.5 * (1.0 - jnp.cos(jnp.pi * jnp.clip(alpha - k, 0.0, 1.0)))


def kernel(points, iter_nr, table, shifts, W0, b0, W1, b1, W2, b2, W3, b3):
    window = _c2f_window(iter_nr)
    pts_pad = jnp.pad(points, ((0, N_PAD - N_POINTS), (0, 0)))
    pts_t = pts_pad.T
    xr = pts_t[0].reshape(R_ROWS, LANES)
    yr = pts_t[1].reshape(R_ROWS, LANES)
    zr = pts_t[2].reshape(R_ROWS, LANES)

    idx_all, w_all = _phase1(shifts, window, xr, yr, zr)
    idx_flat = idx_all.reshape(NV * N_PAD)
    w_flat = w_all.reshape(NV * N_PAD)
    del idx_all, w_all

    table1d = table.reshape(NR_LEVELS * CAPACITY * NR_FEAT)
    feats3 = _phase2(table1d, idx_flat, w_flat).reshape(
        2 * NR_LEVELS, R_ROWS, LANES)

    pts3 = pts_t.reshape(POS_DIM, R_ROWS, LANES)
    out3 = _phase3(feats3, pts3, W0, b0, W1, b1, W2, b2, W3, b3)
    sdf = out3[0].reshape(N_PAD)[:N_POINTS, None]
    geom = out3[1:].reshape(GEOM_FEAT, N_PAD)[:, :N_POINTS].T
    return (sdf, geom)


# speedup vs baseline: 1.0162x; 1.0162x over previous
import functools
import math

import jax
import jax.numpy as jnp
import numpy as np
from jax import lax
from jax.experimental import pallas as pl
from jax.experimental.pallas import tpu as pltpu
from jax.experimental.pallas import tpu_sc as plsc

POS_DIM = 3
CAPACITY = 2 ** 18
NR_LEVELS = 24
NR_FEAT = 2
GEOM_FEAT = 32
NR_ITERS_C2F = 10000
SCALES = np.geomspace(1.0, 0.0001, NR_LEVELS).astype(np.float32)
P0 = np.int32(np.uint32(2654435761).astype(np.int64) - (1 << 32))
P1 = np.int32(805459861)
P2 = np.int32(np.uint32(3674653429).astype(np.int64) - (1 << 32))
SF = [(POS_DIM + 1) / math.sqrt((i + 1) * (i + 2)) for i in range(POS_DIM)]

N_POINTS = 200000
LANES = 128
N_PAD = 200704
R_ROWS = N_PAD // LANES
RB = 8
NV = 96
SC_TILES = 32
PT_PER_TILE = N_PAD // SC_TILES
SC_CHUNK = 128
SC_NCHUNKS = PT_PER_TILE // SC_CHUNK
B3 = 2048


def _lattice_body(shifts_ref, window_ref, x_ref, y_ref, z_ref, idx_ref, w_ref):
    x = x_ref[...]
    y = y_ref[...]
    z = z_ref[...]
    for l in range(NR_LEVELS):
        scale = float(SCALES[l])
        xs0 = (x + shifts_ref[l, 0]) / scale * SF[0]
        xs1 = (y + shifts_ref[l, 1]) / scale * SF[1]
        xs2 = (z + shifts_ref[l, 2]) / scale * SF[2]
        e3 = -3.0 * xs2
        sm = xs2
        e2 = sm - 2.0 * xs1
        sm = sm + xs1
        e1 = sm - xs0
        sm = sm + xs0
        e0 = sm
        elev = [e0, e1, e2, e3]
        rem0 = [jnp.floor(e / 4.0 + 0.5) * 4.0 for e in elev]
        diff = [elev[j] - rem0[j] for j in range(4)]
        rank = [jnp.zeros_like(x, jnp.int32) for _ in range(4)]
        for i in range(4):
            for j in range(i + 1, 4):
                c = (diff[i] < diff[j]).astype(jnp.int32)
                rank[i] = rank[i] + c
                rank[j] = rank[j] + (1 - c)
        ssum = ((rem0[0] + rem0[1]) + (rem0[2] + rem0[3])) / 4.0
        s_i = jnp.floor(ssum + 0.5).astype(jnp.int32)
        for j in range(4):
            rank[j] = rank[j] + s_i
            low = (rank[j] < 0).astype(jnp.int32)
            high = (rank[j] > 3).astype(jnp.int32)
            rank[j] = rank[j] + 4 * low - 4 * high
            rem0[j] = rem0[j] + 4.0 * (low - high).astype(jnp.float32)
        v = [(elev[j] - rem0[j]) / 4.0 for j in range(4)]
        bc = []
        for k in range(5):
            acc = jnp.zeros_like(x)
            for j in range(4):
                acc = acc + v[j] * ((rank[j] == 3 - k).astype(jnp.float32)
                                    - (rank[j] == 4 - k).astype(jnp.float32))
            bc.append(acc)
        bc[0] = bc[0] + (1.0 + bc[4])
        rem0i = [rem0[j].astype(jnp.int32) for j in range(3)]
        win = window_ref[l]
        for k in range(4):
            key0 = rem0i[0] + k - 4 * (rank[0] > 3 - k).astype(jnp.int32)
            key1 = rem0i[1] + k - 4 * (rank[1] > 3 - k).astype(jnp.int32)
            key2 = rem0i[2] + k - 4 * (rank[2] > 3 - k).astype(jnp.int32)
            h = (key0 * P0) ^ (key1 * P1) ^ (key2 * P2)
            idx = ((h & jnp.int32(CAPACITY - 1)) + jnp.int32(l * CAPACITY)) * 2
            idx_ref[0, l * 4 + k] = idx
            w_ref[0, l * 4 + k] = bc[k] * win


def _phase1(shifts, window, xr, yr, zr):
    smem = pl.BlockSpec(memory_space=pltpu.SMEM)
    blk = pl.BlockSpec((RB, LANES), lambda i: (i, 0))
    return pl.pallas_call(
        _lattice_body,
        grid=(R_ROWS // RB,),
        in_specs=[smem, smem, blk, blk, blk],
        out_specs=[pl.BlockSpec((1, NV, RB, LANES), lambda i: (i, 0, 0, 0)),
                   pl.BlockSpec((1, NV, RB, LANES), lambda i: (i, 0, 0, 0))],
        out_shape=[
            jax.ShapeDtypeStruct((R_ROWS // RB, NV, RB, LANES), jnp.int32),
            jax.ShapeDtypeStruct((R_ROWS // RB, NV, RB, LANES), jnp.float32)],
    )(shifts, window, xr, yr, zr)


def _sc_gather_body(table_hbm1d, idx_hbm, w_hbm, out_hbm,
                    idx_v, odd_v, w_v, rows_a_v, rows_b_v, feats_v, sem):
    wid = lax.axis_index("s") * 2 + lax.axis_index("c")
    lane = lax.iota(jnp.int32, 16)

    def chunk_body(g, carry):
        cg = wid * SC_NCHUNKS + g
        base = cg * SC_CHUNK
        i_blk = cg // RB
        s_sub = cg % RB
        row0 = i_blk * (NV * RB * LANES) + s_sub * LANES

        def slab_fire(r, c2):
            off = row0 + r * (RB * LANES)
            pltpu.async_copy(idx_hbm.at[pl.ds(off, SC_CHUNK)],
                             idx_v.at[r], sem)
            pltpu.async_copy(w_hbm.at[pl.ds(off, SC_CHUNK)],
                             w_v.at[r], sem)
            return c2

        lax.fori_loop(0, NV, slab_fire, 0)

        def slab_drain(r, c2):
            pltpu.make_async_copy(idx_hbm.at[pl.ds(0, SC_CHUNK)],
                                  idx_v.at[0], sem).wait()
            pltpu.make_async_copy(w_hbm.at[pl.ds(0, SC_CHUNK)],
                                  w_v.at[0], sem).wait()
            return c2

        lax.fori_loop(0, NV, slab_drain, 0)

        def mk_odd(r, c2):
            for v in range(SC_CHUNK // 16):
                odd_v[r, pl.ds(v * 16, 16)] = idx_v[r, pl.ds(v * 16, 16)] + 1
            return c2

        lax.fori_loop(0, NV, mk_odd, 0)

        def fire(r, c2):
            pltpu.async_copy(table_hbm1d.at[idx_v.at[r]], rows_a_v.at[r], sem)
            pltpu.async_copy(table_hbm1d.at[odd_v.at[r]], rows_b_v.at[r], sem)
            return c2

        lax.fori_loop(0, NV, fire, 0)

        def drain(r, c2):
            pltpu.make_async_copy(table_hbm1d.at[idx_v.at[0]],
                                  rows_a_v.at[0], sem).wait()
            pltpu.make_async_copy(table_hbm1d.at[idx_v.at[0]],
                                  rows_b_v.at[0], sem).wait()
            return c2

        lax.fori_loop(0, NV, drain, 0)

        def level_body(l, carry2):
            for pv in range(SC_CHUNK // 16):
                p0 = pv * 16
                for c in range(2):
                    rows_cv = rows_a_v if c == 0 else rows_b_v
                    acc = jnp.zeros((16,), jnp.float32)
                    for k in range(4):
                        r = 4 * l + k
                        wk = w_v[r, pl.ds(p0, 16)]
                        rvals = rows_cv[r, pl.ds(p0, 16)]
                        acc = acc + wk * rvals
                    feats_v[2 * l + c, pl.ds(p0, 16)] = acc
            return carry2

        lax.fori_loop(0, NR_LEVELS, level_body, 0)

        def out_fire(f, c2):
            pltpu.async_copy(feats_v.at[f],
                             out_hbm.at[pl.ds(f * N_PAD + base, SC_CHUNK)], sem)
            return c2

        lax.fori_loop(0, 2 * NR_LEVELS, out_fire, 0)

        def out_drain(f, c2):
            pltpu.make_async_copy(feats_v.at[0],
                                  out_hbm.at[pl.ds(0, SC_CHUNK)], sem).wait()
            return c2

        lax.fori_loop(0, 2 * NR_LEVELS, out_drain, 0)
        return carry

    lax.fori_loop(0, SC_NCHUNKS, chunk_body, 0)


def _phase2(table2, idx_all, w_all):
    mesh = plsc.VectorSubcoreMesh(core_axis_name="c", subcore_axis_name="s")
    kern = functools.partial(
        pl.kernel,
        out_type=jax.ShapeDtypeStruct((N_PAD * 2 * NR_LEVELS,), jnp.float32),
        mesh=mesh,
        scratch_types=[
            pltpu.VMEM((NV, SC_CHUNK), jnp.int32),
            pltpu.VMEM((NV, SC_CHUNK), jnp.int32),
            pltpu.VMEM((NV, SC_CHUNK), jnp.float32),
            pltpu.VMEM((NV, SC_CHUNK), jnp.float32),
            pltpu.VMEM((NV, SC_CHUNK), jnp.float32),
            pltpu.VMEM((2 * NR_LEVELS, SC_CHUNK), jnp.float32),
            pltpu.SemaphoreType.DMA,
        ],
        compiler_params=pltpu.CompilerParams(needs_layout_passes=False,
                                             use_tc_tiling_on_sc=False),
    )(_sc_gather_body)
    return kern(table2, idx_all, w_all)


def _gelu(x):
    return 0.5 * x * (1.0 + lax.erf(x * np.float32(1.0 / math.sqrt(2.0))))


RB3 = 16


def _mlp_body(feats_ref, pts_ref, w0f_ref, w0p_ref, b0_ref, w1_ref, b1_ref,
              w2_ref, b2_ref, w3_ref, b3_ref, out_ref):
    w0f = w0f_ref[...]
    w0p = w0p_ref[...]
    w1 = w1_ref[...]
    w2 = w2_ref[...]
    w3 = w3_ref[...]
    for j in range(RB3):
        f = feats_ref[:, j, :]
        p = pts_ref[:, j, :] * 0.001
        h = jnp.dot(w0f, f, preferred_element_type=jnp.float32)
        h = h + jnp.dot(w0p, p, preferred_element_type=jnp.float32)
        h = _gelu(h + b0_ref[...])
        h = _gelu(jnp.dot(w1, h, preferred_element_type=jnp.float32)
                  + b1_ref[...])
        h = _gelu(jnp.dot(w2, h, preferred_element_type=jnp.float32)
                  + b2_ref[...])
        res = lax.dot_general(h, w3, (((0,), (0,)), ((), ())),
                              preferred_element_type=jnp.float32)
        out_ref[pl.ds(j * LANES, LANES), :] = res + b3_ref[...]


def _phase3(feats3, pts3, W0, b0, W1, b1, W2, b2, W3, b3):
    def whole(shape):
        return pl.BlockSpec(shape, lambda i: (0, 0))
    nf = 2 * NR_LEVELS
    no = 1 + GEOM_FEAT
    return pl.pallas_call(
        _mlp_body,
        grid=(R_ROWS // RB3,),
        in_specs=[pl.BlockSpec((nf, RB3, LANES), lambda i: (0, i, 0)),
                  pl.BlockSpec((POS_DIM, RB3, LANES), lambda i: (0, i, 0)),
                  whole((32, nf)), whole((32, POS_DIM)),
                  whole((32, 1)), whole((32, 32)), whole((32, 1)),
                  whole((32, 32)), whole((32, 1)),
                  whole((32, no)), whole((1, no))],
        out_specs=pl.BlockSpec((RB3 * LANES, no), lambda i: (i, 0)),
        out_shape=jax.ShapeDtypeStruct((N_PAD, no), jnp.float32),
    )(feats3, pts3,
      W0[:nf].T, W0[nf:].T, b0.reshape(-1, 1),
      W1.T, b1.reshape(-1, 1), W2.T, b2.reshape(-1, 1),
      W3, b3.reshape(1, -1))


def _c2f_window(iter_nr):
    t = jnp.clip(jnp.asarray(iter_nr, jnp.float32), 0.0, float(NR_ITERS_C2F))
    tt = 0.3 + (1.0 - 0.3) * t / float(NR_ITERS_C2F)
    alpha = tt * NR_LEVELS
    k = jnp.arange(NR_LEVELS, dtype=jnp.float32)
    return 0.5 * (1.0 - jnp.cos(jnp.pi * jnp.clip(alpha - k, 0.0, 1.0)))


def kernel(points, iter_nr, table, shifts, W0, b0, W1, b1, W2, b2, W3, b3):
    window = _c2f_window(iter_nr)
    pts_pad = jnp.pad(points, ((0, N_PAD - N_POINTS), (0, 0)))
    pts_t = pts_pad.T
    xr = pts_t[0].reshape(R_ROWS, LANES)
    yr = pts_t[1].reshape(R_ROWS, LANES)
    zr = pts_t[2].reshape(R_ROWS, LANES)

    idx_all, w_all = _phase1(shifts, window, xr, yr, zr)
    idx_flat = idx_all.reshape(NV * N_PAD)
    w_flat = w_all.reshape(NV * N_PAD)
    del idx_all, w_all

    table1d = table.reshape(NR_LEVELS * CAPACITY * NR_FEAT)
    feats3 = _phase2(table1d, idx_flat, w_flat).reshape(
        2 * NR_LEVELS, R_ROWS, LANES)

    pts3 = pts_t.reshape(POS_DIM, R_ROWS, LANES)
    out = _phase3(feats3, pts3, W0, b0, W1, b1, W2, b2, W3, b3)
    sdf = out[:N_POINTS, 0:1]
    geom = out[:N_POINTS, 1:]
    return (sdf, geom)


# speedup vs baseline: 4.0801x; 4.0151x over previous
import functools
import math

import jax
import jax.numpy as jnp
import numpy as np
from jax import lax
from jax.experimental import pallas as pl
from jax.experimental.pallas import tpu as pltpu
from jax.experimental.pallas import tpu_sc as plsc

POS_DIM = 3
CAPACITY = 2 ** 18
NR_LEVELS = 24
NR_FEAT = 2
GEOM_FEAT = 32
NR_ITERS_C2F = 10000
SCALES = np.geomspace(1.0, 0.0001, NR_LEVELS).astype(np.float32)
P0 = np.int32(np.uint32(2654435761).astype(np.int64) - (1 << 32))
P1 = np.int32(805459861)
P2 = np.int32(np.uint32(3674653429).astype(np.int64) - (1 << 32))
SF = [(POS_DIM + 1) / math.sqrt((i + 1) * (i + 2)) for i in range(POS_DIM)]

N_POINTS = 200000
LANES = 128
N_PAD = 200704
R_ROWS = N_PAD // LANES
RB = 8
NV = 96
SC_TILES = 32
PT_PER_TILE = N_PAD // SC_TILES
SC_CHUNK = 128
SC_NCHUNKS = PT_PER_TILE // SC_CHUNK
B3 = 2048


def _lattice_body(shifts_ref, window_ref, x_ref, y_ref, z_ref, idx_ref, w_ref):
    x = x_ref[...]
    y = y_ref[...]
    z = z_ref[...]
    for l in range(NR_LEVELS):
        scale = float(SCALES[l])
        xs0 = (x + shifts_ref[l, 0]) / scale * SF[0]
        xs1 = (y + shifts_ref[l, 1]) / scale * SF[1]
        xs2 = (z + shifts_ref[l, 2]) / scale * SF[2]
        e3 = -3.0 * xs2
        sm = xs2
        e2 = sm - 2.0 * xs1
        sm = sm + xs1
        e1 = sm - xs0
        sm = sm + xs0
        e0 = sm
        elev = [e0, e1, e2, e3]
        rem0 = [jnp.floor(e / 4.0 + 0.5) * 4.0 for e in elev]
        diff = [elev[j] - rem0[j] for j in range(4)]
        rank = [jnp.zeros_like(x, jnp.int32) for _ in range(4)]
        for i in range(4):
            for j in range(i + 1, 4):
                c = (diff[i] < diff[j]).astype(jnp.int32)
                rank[i] = rank[i] + c
                rank[j] = rank[j] + (1 - c)
        ssum = ((rem0[0] + rem0[1]) + (rem0[2] + rem0[3])) / 4.0
        s_i = jnp.floor(ssum + 0.5).astype(jnp.int32)
        for j in range(4):
            rank[j] = rank[j] + s_i
            low = (rank[j] < 0).astype(jnp.int32)
            high = (rank[j] > 3).astype(jnp.int32)
            rank[j] = rank[j] + 4 * low - 4 * high
            rem0[j] = rem0[j] + 4.0 * (low - high).astype(jnp.float32)
        v = [(elev[j] - rem0[j]) / 4.0 for j in range(4)]
        bc = []
        for k in range(5):
            acc = jnp.zeros_like(x)
            for j in range(4):
                acc = acc + v[j] * ((rank[j] == 3 - k).astype(jnp.float32)
                                    - (rank[j] == 4 - k).astype(jnp.float32))
            bc.append(acc)
        bc[0] = bc[0] + (1.0 + bc[4])
        rem0i = [rem0[j].astype(jnp.int32) for j in range(3)]
        win = window_ref[l]
        for k in range(4):
            key0 = rem0i[0] + k - 4 * (rank[0] > 3 - k).astype(jnp.int32)
            key1 = rem0i[1] + k - 4 * (rank[1] > 3 - k).astype(jnp.int32)
            key2 = rem0i[2] + k - 4 * (rank[2] > 3 - k).astype(jnp.int32)
            h = (key0 * P0) ^ (key1 * P1) ^ (key2 * P2)
            e = h & jnp.int32(CAPACITY - 1)
            idx = ((jnp.int32(l * 2 * CAPACITY)
                    + ((e >> 7) << 8)) + (e & jnp.int32(127)))
            idx_ref[0, l * 4 + k] = idx
            w_ref[0, l * 4 + k] = bc[k] * win


def _phase1(shifts, window, xr, yr, zr):
    smem = pl.BlockSpec(memory_space=pltpu.SMEM)
    blk = pl.BlockSpec((RB, LANES), lambda i: (i, 0))
    return pl.pallas_call(
        _lattice_body,
        grid=(R_ROWS // RB,),
        in_specs=[smem, smem, blk, blk, blk],
        out_specs=[pl.BlockSpec((1, NV, RB, LANES), lambda i: (i, 0, 0, 0)),
                   pl.BlockSpec((1, NV, RB, LANES), lambda i: (i, 0, 0, 0))],
        out_shape=[
            jax.ShapeDtypeStruct((R_ROWS // RB, NV, RB, LANES), jnp.int32),
            jax.ShapeDtypeStruct((R_ROWS // RB, NV, RB, LANES), jnp.float32)],
    )(shifts, window, xr, yr, zr)


def _sc_gather_body(table_hbm1d, idx_hbm, w_hbm, out_hbm,
                    idx_v, odd_v, w_v, rows_a_v, rows_b_v, feats_v, sem):
    wid = lax.axis_index("s") * 2 + lax.axis_index("c")
    lane = lax.iota(jnp.int32, 16)

    def chunk_body(g, carry):
        cg = wid * SC_NCHUNKS + g
        base = cg * SC_CHUNK
        i_blk = cg // RB
        s_sub = cg % RB
        row0 = i_blk * (NV * RB * LANES) + s_sub * LANES

        def slab_fire(r, c2):
            off = row0 + r * (RB * LANES)
            pltpu.async_copy(idx_hbm.at[pl.ds(off, SC_CHUNK)],
                             idx_v.at[r], sem)
            pltpu.async_copy(w_hbm.at[pl.ds(off, SC_CHUNK)],
                             w_v.at[r], sem)
            return c2

        lax.fori_loop(0, NV, slab_fire, 0)

        def slab_drain(r, c2):
            pltpu.make_async_copy(idx_hbm.at[pl.ds(0, SC_CHUNK)],
                                  idx_v.at[0], sem).wait()
            pltpu.make_async_copy(w_hbm.at[pl.ds(0, SC_CHUNK)],
                                  w_v.at[0], sem).wait()
            return c2

        lax.fori_loop(0, NV, slab_drain, 0)

        def mk_odd(r, c2):
            for v in range(SC_CHUNK // 16):
                odd_v[r, pl.ds(v * 16, 16)] = idx_v[r, pl.ds(v * 16, 16)] + 128
            return c2

        lax.fori_loop(0, NV, mk_odd, 0)

        def fire(r, c2):
            pltpu.async_copy(table_hbm1d.at[idx_v.at[r]], rows_a_v.at[r], sem)
            pltpu.async_copy(table_hbm1d.at[odd_v.at[r]], rows_b_v.at[r], sem)
            return c2

        lax.fori_loop(0, NV, fire, 0)

        def drain(r, c2):
            pltpu.make_async_copy(table_hbm1d.at[idx_v.at[0]],
                                  rows_a_v.at[0], sem).wait()
            pltpu.make_async_copy(table_hbm1d.at[idx_v.at[0]],
                                  rows_b_v.at[0], sem).wait()
            return c2

        lax.fori_loop(0, NV, drain, 0)

        def level_body(l, carry2):
            for pv in range(SC_CHUNK // 16):
                p0 = pv * 16
                for c in range(2):
                    rows_cv = rows_a_v if c == 0 else rows_b_v
                    acc = jnp.zeros((16,), jnp.float32)
                    for k in range(4):
                        r = 4 * l + k
                        wk = w_v[r, pl.ds(p0, 16)]
                        rvals = rows_cv[r, pl.ds(p0, 16)]
                        acc = acc + wk * rvals
                    feats_v[2 * l + c, pl.ds(p0, 16)] = acc
            return carry2

        lax.fori_loop(0, NR_LEVELS, level_body, 0)

        def out_fire(f, c2):
            pltpu.async_copy(feats_v.at[f],
                             out_hbm.at[pl.ds(f * N_PAD + base, SC_CHUNK)], sem)
            return c2

        lax.fori_loop(0, 2 * NR_LEVELS, out_fire, 0)

        def out_drain(f, c2):
            pltpu.make_async_copy(feats_v.at[0],
                                  out_hbm.at[pl.ds(0, SC_CHUNK)], sem).wait()
            return c2

        lax.fori_loop(0, 2 * NR_LEVELS, out_drain, 0)
        return carry

    lax.fori_loop(0, SC_NCHUNKS, chunk_body, 0)


def _phase2(table2, idx_all, w_all):
    mesh = plsc.VectorSubcoreMesh(core_axis_name="c", subcore_axis_name="s")
    kern = functools.partial(
        pl.kernel,
        out_type=jax.ShapeDtypeStruct((N_PAD * 2 * NR_LEVELS,), jnp.float32),
        mesh=mesh,
        scratch_types=[
            pltpu.VMEM((NV, SC_CHUNK), jnp.int32),
            pltpu.VMEM((NV, SC_CHUNK), jnp.int32),
            pltpu.VMEM((NV, SC_CHUNK), jnp.float32),
            pltpu.VMEM((NV, SC_CHUNK), jnp.float32),
            pltpu.VMEM((NV, SC_CHUNK), jnp.float32),
            pltpu.VMEM((2 * NR_LEVELS, SC_CHUNK), jnp.float32),
            pltpu.SemaphoreType.DMA,
        ],
        compiler_params=pltpu.CompilerParams(needs_layout_passes=False,
                                             use_tc_tiling_on_sc=False),
    )(_sc_gather_body)
    return kern(table2, idx_all, w_all)


def _gelu(x):
    return 0.5 * x * (1.0 + lax.erf(x * np.float32(1.0 / math.sqrt(2.0))))


RB3 = 16


def _mlp_body(feats_ref, pts_ref, w0f_ref, w0p_ref, b0_ref, w1_ref, b1_ref,
              w2_ref, b2_ref, w3_ref, b3_ref, out_ref):
    w0f = w0f_ref[...]
    w0p = w0p_ref[...]
    w1 = w1_ref[...]
    w2 = w2_ref[...]
    w3 = w3_ref[...]
    for j in range(RB3):
        f = feats_ref[:, j, :]
        p = pts_ref[:, j, :] * 0.001
        h = jnp.dot(w0f, f, preferred_element_type=jnp.float32)
        h = h + jnp.dot(w0p, p, preferred_element_type=jnp.float32)
        h = _gelu(h + b0_ref[...])
        h = _gelu(jnp.dot(w1, h, preferred_element_type=jnp.float32)
                  + b1_ref[...])
        h = _gelu(jnp.dot(w2, h, preferred_element_type=jnp.float32)
                  + b2_ref[...])
        res = lax.dot_general(h, w3, (((0,), (0,)), ((), ())),
                              preferred_element_type=jnp.float32)
        out_ref[pl.ds(j * LANES, LANES), :] = res + b3_ref[...]


def _phase3(feats3, pts3, W0, b0, W1, b1, W2, b2, W3, b3):
    def whole(shape):
        return pl.BlockSpec(shape, lambda i: (0, 0))
    nf = 2 * NR_LEVELS
    no = 1 + GEOM_FEAT
    return pl.pallas_call(
        _mlp_body,
        grid=(R_ROWS // RB3,),
        in_specs=[pl.BlockSpec((nf, RB3, LANES), lambda i: (0, i, 0)),
                  pl.BlockSpec((POS_DIM, RB3, LANES), lambda i: (0, i, 0)),
                  whole((32, nf)), whole((32, POS_DIM)),
                  whole((32, 1)), whole((32, 32)), whole((32, 1)),
                  whole((32, 32)), whole((32, 1)),
                  whole((32, no)), whole((1, no))],
        out_specs=pl.BlockSpec((RB3 * LANES, no), lambda i: (i, 0)),
        out_shape=jax.ShapeDtypeStruct((N_PAD, no), jnp.float32),
    )(feats3, pts3,
      W0[:nf].T, W0[nf:].T, b0.reshape(-1, 1),
      W1.T, b1.reshape(-1, 1), W2.T, b2.reshape(-1, 1),
      W3, b3.reshape(1, -1))


def _c2f_window(iter_nr):
    t = jnp.clip(jnp.asarray(iter_nr, jnp.float32), 0.0, float(NR_ITERS_C2F))
    tt = 0.3 + (1.0 - 0.3) * t / float(NR_ITERS_C2F)
    alpha = tt * NR_LEVELS
    k = jnp.arange(NR_LEVELS, dtype=jnp.float32)
    return 0.5 * (1.0 - jnp.cos(jnp.pi * jnp.clip(alpha - k, 0.0, 1.0)))


def kernel(points, iter_nr, table, shifts, W0, b0, W1, b1, W2, b2, W3, b3):
    window = _c2f_window(iter_nr)
    pts_pad = jnp.pad(points, ((0, N_PAD - N_POINTS), (0, 0)))
    pts_t = pts_pad.T
    xr = pts_t[0].reshape(R_ROWS, LANES)
    yr = pts_t[1].reshape(R_ROWS, LANES)
    zr = pts_t[2].reshape(R_ROWS, LANES)

    idx_all, w_all = _phase1(shifts, window, xr, yr, zr)
    idx_flat = idx_all.reshape(NV * N_PAD)
    w_flat = w_all.reshape(NV * N_PAD)
    del idx_all, w_all

    table1d = table.reshape(
        NR_LEVELS, CAPACITY // 128, 128, NR_FEAT).transpose(
        0, 1, 3, 2).reshape(NR_LEVELS * CAPACITY * NR_FEAT)
    feats3 = _phase2(table1d, idx_flat, w_flat).reshape(
        2 * NR_LEVELS, R_ROWS, LANES)

    pts3 = pts_t.reshape(POS_DIM, R_ROWS, LANES)
    out = _phase3(feats3, pts3, W0, b0, W1, b1, W2, b2, W3, b3)
    sdf = out[:N_POINTS, 0:1]
    geom = out[:N_POINTS, 1:]
    return (sdf, geom)
